# baseline (speedup 1.0000x reference)
import jax
import jax.numpy as jnp
from jax import lax
from jax.experimental import pallas as pl
from jax.experimental.pallas import tpu as pltpu
from jax.experimental.pallas import tpu_sc as plsc

NUM_CORES = 2
NUM_SUBCORES = 16
NUM_WORKERS = NUM_CORES * NUM_SUBCORES
LANES = 16

BATCH = 16384
HIDDEN = 128
ROWS_PER_WORKER = BATCH // NUM_WORKERS
CHUNK = 128
NUM_CHUNKS = ROWS_PER_WORKER // CHUNK
GROUPS_PER_CHUNK = CHUNK // LANES


def _mf_body(user_ids, item_ids, user_table, item_table, out_hbm,
             uidx, iidx, urows, irows, out_v,
             sem_uidx, sem_iidx, sem_urows, sem_irows):
  wid = lax.axis_index("s") * NUM_CORES + lax.axis_index("c")
  base = wid * ROWS_PER_WORKER

  def start_idx(c, p):
    off = base + c * CHUNK
    pltpu.async_copy(user_ids.at[pl.ds(off, CHUNK)], uidx.at[p], sem_uidx.at[p])
    pltpu.async_copy(item_ids.at[pl.ds(off, CHUNK)], iidx.at[p], sem_iidx.at[p])

  def wait_idx(c, p):
    off = base + c * CHUNK
    pltpu.make_async_copy(user_ids.at[pl.ds(off, CHUNK)], uidx.at[p],
                          sem_uidx.at[p]).wait()
    pltpu.make_async_copy(item_ids.at[pl.ds(off, CHUNK)], iidx.at[p],
                          sem_iidx.at[p]).wait()

  def start_rows(p):
    pltpu.async_copy(user_table.at[uidx.at[p]], urows.at[p], sem_urows.at[p])
    pltpu.async_copy(item_table.at[iidx.at[p]], irows.at[p], sem_irows.at[p])

  def wait_rows(p):
    pltpu.make_async_copy(user_table.at[uidx.at[p]], urows.at[p],
                          sem_urows.at[p]).wait()
    pltpu.make_async_copy(item_table.at[iidx.at[p]], irows.at[p],
                          sem_irows.at[p]).wait()

  lane_id = lax.iota(jnp.int32, LANES)

  def compute_chunk(c, p):
    pv = jnp.full((LANES,), p, jnp.int32)

    def group_body(g, _):
      rowv = g * LANES + lane_id

      def dot_body(d4, scores):
        for jj in range(4):
          colv = jnp.full((LANES,), d4 * 4 + jj, jnp.int32)
          gu = plsc.load_gather(urows, [pv, rowv, colv])
          gi = plsc.load_gather(irows, [pv, rowv, colv])
          scores = scores + gu * gi
        return scores

      scores = lax.fori_loop(0, HIDDEN // 4, dot_body,
                             jnp.zeros((LANES,), jnp.float32))
      out_v[pl.ds(c * CHUNK + g * LANES, LANES)] = scores
      return 0

    lax.fori_loop(0, GROUPS_PER_CHUNK, group_body, 0)

  start_idx(0, 0)
  for c in range(NUM_CHUNKS):
    p = c % 2
    wait_idx(c, p)
    start_rows(p)
    if c + 1 < NUM_CHUNKS:
      start_idx(c + 1, (c + 1) % 2)
    if c > 0:
      wait_rows((c - 1) % 2)
      compute_chunk(c - 1, (c - 1) % 2)
  wait_rows((NUM_CHUNKS - 1) % 2)
  compute_chunk(NUM_CHUNKS - 1, (NUM_CHUNKS - 1) % 2)

  pltpu.sync_copy(out_v, out_hbm.at[pl.ds(base, ROWS_PER_WORKER)])


@jax.jit
def _mf_scores(batch_user_ids, batch_item_ids, user_table, item_table):
  mesh = plsc.VectorSubcoreMesh(
      core_axis_name="c", subcore_axis_name="s",
      num_cores=NUM_CORES, num_subcores=NUM_SUBCORES)
  grid_kernel = pl.kernel(
      _mf_body,
      out_type=jax.ShapeDtypeStruct((BATCH,), jnp.float32),
      mesh=mesh,
      compiler_params=pltpu.CompilerParams(needs_layout_passes=False),
      scratch_types=[
          pltpu.VMEM((2, CHUNK), jnp.int32),
          pltpu.VMEM((2, CHUNK), jnp.int32),
          pltpu.VMEM((2, CHUNK, HIDDEN), jnp.float32),
          pltpu.VMEM((2, CHUNK, HIDDEN), jnp.float32),
          pltpu.VMEM((ROWS_PER_WORKER,), jnp.float32),
          pltpu.SemaphoreType.DMA((2,)),
          pltpu.SemaphoreType.DMA((2,)),
          pltpu.SemaphoreType.DMA((2,)),
          pltpu.SemaphoreType.DMA((2,)),
      ],
  )
  return grid_kernel(batch_user_ids, batch_item_ids, user_table, item_table)


def kernel(batch_user_ids, batch_item_ids, user_table, item_table):
  return _mf_scores(batch_user_ids, batch_item_ids, user_table, item_table)


# speedup vs baseline: 1.7547x; 1.7547x over previous
import jax
import jax.numpy as jnp
from jax import lax
from jax.experimental import pallas as pl
from jax.experimental.pallas import tpu as pltpu
from jax.experimental.pallas import tpu_sc as plsc

NUM_CORES = 2
NUM_SUBCORES = 16
NUM_WORKERS = NUM_CORES * NUM_SUBCORES
LANES = 16

BATCH = 16384
HIDDEN = 128
ROWS_PER_WORKER = BATCH // NUM_WORKERS
CHUNK = 128
NUM_CHUNKS = ROWS_PER_WORKER // CHUNK
GROUPS_PER_CHUNK = CHUNK // LANES


def _mf_body(user_ids, item_ids, user_table, item_table, out_hbm,
             uidx, iidx, urows, irows, out_v,
             sem_uidx, sem_iidx, sem_urows, sem_irows):
  wid = lax.axis_index("s") * NUM_CORES + lax.axis_index("c")
  base = wid * ROWS_PER_WORKER

  def start_idx(c, p):
    off = base + c * CHUNK
    pltpu.async_copy(user_ids.at[pl.ds(off, CHUNK)], uidx.at[p], sem_uidx.at[p])
    pltpu.async_copy(item_ids.at[pl.ds(off, CHUNK)], iidx.at[p], sem_iidx.at[p])

  def wait_idx(c, p):
    off = base + c * CHUNK
    pltpu.make_async_copy(user_ids.at[pl.ds(off, CHUNK)], uidx.at[p],
                          sem_uidx.at[p]).wait()
    pltpu.make_async_copy(item_ids.at[pl.ds(off, CHUNK)], iidx.at[p],
                          sem_iidx.at[p]).wait()

  def start_rows(p):
    pltpu.async_copy(user_table.at[uidx.at[p]], urows.at[p], sem_urows.at[p])
    pltpu.async_copy(item_table.at[iidx.at[p]], irows.at[p], sem_irows.at[p])

  def wait_rows(p):
    pltpu.make_async_copy(user_table.at[uidx.at[p]], urows.at[p],
                          sem_urows.at[p]).wait()
    pltpu.make_async_copy(item_table.at[iidx.at[p]], irows.at[p],
                          sem_irows.at[p]).wait()

  lane_id = lax.iota(jnp.int32, LANES)

  def compute_chunk(c, p):
    def group_body(g, _):
      scores = jnp.zeros((LANES,), jnp.float32)
      for k in range(LANES):
        r = g * LANES + k
        acc = urows[p, r, pl.ds(0, LANES)] * irows[p, r, pl.ds(0, LANES)]
        for j in range(1, HIDDEN // LANES):
          acc = acc + (urows[p, r, pl.ds(j * LANES, LANES)] *
                       irows[p, r, pl.ds(j * LANES, LANES)])
        s = jnp.sum(acc)
        scores = jnp.where(lane_id == k, s, scores)
      out_v[pl.ds(c * CHUNK + g * LANES, LANES)] = scores
      return 0

    lax.fori_loop(0, GROUPS_PER_CHUNK, group_body, 0)

  start_idx(0, 0)
  for c in range(NUM_CHUNKS):
    p = c % 2
    wait_idx(c, p)
    start_rows(p)
    if c + 1 < NUM_CHUNKS:
      start_idx(c + 1, (c + 1) % 2)
    if c > 0:
      wait_rows((c - 1) % 2)
      compute_chunk(c - 1, (c - 1) % 2)
  wait_rows((NUM_CHUNKS - 1) % 2)
  compute_chunk(NUM_CHUNKS - 1, (NUM_CHUNKS - 1) % 2)

  pltpu.sync_copy(out_v, out_hbm.at[pl.ds(base, ROWS_PER_WORKER)])


@jax.jit
def _mf_scores(batch_user_ids, batch_item_ids, user_table, item_table):
  mesh = plsc.VectorSubcoreMesh(
      core_axis_name="c", subcore_axis_name="s",
      num_cores=NUM_CORES, num_subcores=NUM_SUBCORES)
  grid_kernel = pl.kernel(
      _mf_body,
      out_type=jax.ShapeDtypeStruct((BATCH,), jnp.float32),
      mesh=mesh,
      compiler_params=pltpu.CompilerParams(needs_layout_passes=False),
      scratch_types=[
          pltpu.VMEM((2, CHUNK), jnp.int32),
          pltpu.VMEM((2, CHUNK), jnp.int32),
          pltpu.VMEM((2, CHUNK, HIDDEN), jnp.float32),
          pltpu.VMEM((2, CHUNK, HIDDEN), jnp.float32),
          pltpu.VMEM((ROWS_PER_WORKER,), jnp.float32),
          pltpu.SemaphoreType.DMA((2,)),
          pltpu.SemaphoreType.DMA((2,)),
          pltpu.SemaphoreType.DMA((2,)),
          pltpu.SemaphoreType.DMA((2,)),
      ],
  )
  return grid_kernel(batch_user_ids, batch_item_ids, user_table, item_table)


def kernel(batch_user_ids, batch_item_ids, user_table, item_table):
  return _mf_scores(batch_user_ids, batch_item_ids, user_table, item_table)


# speedup vs baseline: 2.0087x; 1.1447x over previous
import jax
import jax.numpy as jnp
from jax import lax
from jax.experimental import pallas as pl
from jax.experimental.pallas import tpu as pltpu
from jax.experimental.pallas import tpu_sc as plsc

NUM_CORES = 2
NUM_SUBCORES = 16
NUM_WORKERS = NUM_CORES * NUM_SUBCORES
LANES = 16

BATCH = 16384
HIDDEN = 128
ROWS_PER_WORKER = BATCH // NUM_WORKERS
CHUNK = 128
NUM_CHUNKS = ROWS_PER_WORKER // CHUNK
GROUPS_PER_CHUNK = CHUNK // LANES


def _mf_body(user_ids, item_ids, user_table, item_table, out_hbm,
             uidx, iidx, urows, irows, out_v,
             sem_uidx, sem_iidx, sem_urows, sem_irows):
  wid = lax.axis_index("s") * NUM_CORES + lax.axis_index("c")
  base = wid * ROWS_PER_WORKER

  def start_idx(c, p):
    off = base + c * CHUNK
    pltpu.async_copy(user_ids.at[pl.ds(off, CHUNK)], uidx.at[p], sem_uidx.at[p])
    pltpu.async_copy(item_ids.at[pl.ds(off, CHUNK)], iidx.at[p], sem_iidx.at[p])

  def wait_idx(c, p):
    off = base + c * CHUNK
    pltpu.make_async_copy(user_ids.at[pl.ds(off, CHUNK)], uidx.at[p],
                          sem_uidx.at[p]).wait()
    pltpu.make_async_copy(item_ids.at[pl.ds(off, CHUNK)], iidx.at[p],
                          sem_iidx.at[p]).wait()

  def start_rows(p):
    pltpu.async_copy(user_table.at[uidx.at[p]], urows.at[p], sem_urows.at[p])
    pltpu.async_copy(item_table.at[iidx.at[p]], irows.at[p], sem_irows.at[p])

  def wait_rows(p):
    pltpu.make_async_copy(user_table.at[uidx.at[p]], urows.at[p],
                          sem_urows.at[p]).wait()
    pltpu.make_async_copy(item_table.at[iidx.at[p]], irows.at[p],
                          sem_irows.at[p]).wait()

  lane_id = lax.iota(jnp.int32, LANES)

  def compute_chunk(c, p):
    @plsc.parallel_loop(0, GROUPS_PER_CHUNK)
    def _(g):
      for k in range(LANES):
        r = g * LANES + k
        prods = [urows[p, r, pl.ds(j * LANES, LANES)] *
                 irows[p, r, pl.ds(j * LANES, LANES)]
                 for j in range(HIDDEN // LANES)]
        while len(prods) > 1:
          prods = [prods[i] + prods[i + 1] for i in range(0, len(prods), 2)]
        rowid = jnp.full((LANES,), c * CHUNK + r, jnp.int32)
        plsc.addupdate_scatter(out_v, [rowid], prods[0])

  @plsc.parallel_loop(0, ROWS_PER_WORKER // LANES)
  def _(i):
    out_v[pl.ds(i * LANES, LANES)] = jnp.zeros((LANES,), jnp.float32)

  start_idx(0, 0)
  for c in range(NUM_CHUNKS):
    p = c % 2
    wait_idx(c, p)
    start_rows(p)
    if c + 1 < NUM_CHUNKS:
      start_idx(c + 1, (c + 1) % 2)
    if c > 0:
      wait_rows((c - 1) % 2)
      compute_chunk(c - 1, (c - 1) % 2)
  wait_rows((NUM_CHUNKS - 1) % 2)
  compute_chunk(NUM_CHUNKS - 1, (NUM_CHUNKS - 1) % 2)

  pltpu.sync_copy(out_v, out_hbm.at[pl.ds(base, ROWS_PER_WORKER)])


@jax.jit
def _mf_scores(batch_user_ids, batch_item_ids, user_table, item_table):
  mesh = plsc.VectorSubcoreMesh(
      core_axis_name="c", subcore_axis_name="s",
      num_cores=NUM_CORES, num_subcores=NUM_SUBCORES)
  grid_kernel = pl.kernel(
      _mf_body,
      out_type=jax.ShapeDtypeStruct((BATCH,), jnp.float32),
      mesh=mesh,
      compiler_params=pltpu.CompilerParams(needs_layout_passes=False),
      scratch_types=[
          pltpu.VMEM((2, CHUNK), jnp.int32),
          pltpu.VMEM((2, CHUNK), jnp.int32),
          pltpu.VMEM((2, CHUNK, HIDDEN), jnp.float32),
          pltpu.VMEM((2, CHUNK, HIDDEN), jnp.float32),
          pltpu.VMEM((ROWS_PER_WORKER,), jnp.float32),
          pltpu.SemaphoreType.DMA((2,)),
          pltpu.SemaphoreType.DMA((2,)),
          pltpu.SemaphoreType.DMA((2,)),
          pltpu.SemaphoreType.DMA((2,)),
      ],
  )
  return grid_kernel(batch_user_ids, batch_item_ids, user_table, item_table)


def kernel(batch_user_ids, batch_item_ids, user_table, item_table):
  return _mf_scores(batch_user_ids, batch_item_ids, user_table, item_table)


# speedup vs baseline: 2.2896x; 1.1398x over previous
import jax
import jax.numpy as jnp
from jax import lax
from jax.experimental import pallas as pl
from jax.experimental.pallas import tpu as pltpu
from jax.experimental.pallas import tpu_sc as plsc

NUM_CORES = 2
NUM_SUBCORES = 16
NUM_WORKERS = NUM_CORES * NUM_SUBCORES
LANES = 16

BATCH = 16384
HIDDEN = 128
ROWS_PER_WORKER = BATCH // NUM_WORKERS
CHUNK = 128
NUM_CHUNKS = ROWS_PER_WORKER // CHUNK
GROUPS_PER_CHUNK = CHUNK // LANES


def _mf_body(user_ids, item_ids, user_table, item_table, out_hbm,
             uidx, iidx, urows, irows, out_v,
             sem_uidx, sem_iidx, sem_urows, sem_irows):
  wid = lax.axis_index("s") * NUM_CORES + lax.axis_index("c")
  base = wid * ROWS_PER_WORKER

  def start_idx(c, p):
    off = base + c * CHUNK
    pltpu.async_copy(user_ids.at[pl.ds(off, CHUNK)], uidx.at[p], sem_uidx.at[p])
    pltpu.async_copy(item_ids.at[pl.ds(off, CHUNK)], iidx.at[p], sem_iidx.at[p])

  def wait_idx(c, p):
    off = base + c * CHUNK
    pltpu.make_async_copy(user_ids.at[pl.ds(off, CHUNK)], uidx.at[p],
                          sem_uidx.at[p]).wait()
    pltpu.make_async_copy(item_ids.at[pl.ds(off, CHUNK)], iidx.at[p],
                          sem_iidx.at[p]).wait()

  def start_rows(p):
    pltpu.async_copy(user_table.at[uidx.at[p]], urows.at[p], sem_urows.at[p])
    pltpu.async_copy(item_table.at[iidx.at[p]], irows.at[p], sem_irows.at[p])

  def wait_rows(p):
    pltpu.make_async_copy(user_table.at[uidx.at[p]], urows.at[p],
                          sem_urows.at[p]).wait()
    pltpu.make_async_copy(item_table.at[iidx.at[p]], irows.at[p],
                          sem_irows.at[p]).wait()

  def compute_chunk(c, p):
    @plsc.parallel_loop(0, CHUNK, unroll=2)
    def _(r):
      prods = [urows[p, r, pl.ds(j * LANES, LANES)] *
               irows[p, r, pl.ds(j * LANES, LANES)]
               for j in range(HIDDEN // LANES)]
      while len(prods) > 1:
        prods = [prods[i] + prods[i + 1] for i in range(0, len(prods), 2)]
      rowid = jnp.full((LANES,), c * CHUNK + r, jnp.int32)
      plsc.addupdate_scatter(out_v, [rowid], prods[0])

  @plsc.parallel_loop(0, ROWS_PER_WORKER // LANES, unroll=4)
  def _(i):
    out_v[pl.ds(i * LANES, LANES)] = jnp.zeros((LANES,), jnp.float32)

  start_idx(0, 0)
  def pipe_body(cc, _):
    c0 = 2 * cc
    wait_idx(c0, 0)
    start_rows(0)

    @pl.when(cc > 0)
    def _():
      wait_rows(1)
      compute_chunk(c0 - 1, 1)

    start_idx(c0 + 1, 1)
    wait_idx(c0 + 1, 1)
    start_rows(1)
    wait_rows(0)
    compute_chunk(c0, 0)

    @pl.when(cc + 1 < NUM_CHUNKS // 2)
    def _():
      start_idx(c0 + 2, 0)

    return 0

  lax.fori_loop(0, NUM_CHUNKS // 2, pipe_body, 0)
  wait_rows(1)
  compute_chunk(NUM_CHUNKS - 1, 1)

  pltpu.sync_copy(out_v, out_hbm.at[pl.ds(base, ROWS_PER_WORKER)])


@jax.jit
def _mf_scores(batch_user_ids, batch_item_ids, user_table, item_table):
  mesh = plsc.VectorSubcoreMesh(
      core_axis_name="c", subcore_axis_name="s",
      num_cores=NUM_CORES, num_subcores=NUM_SUBCORES)
  grid_kernel = pl.kernel(
      _mf_body,
      out_type=jax.ShapeDtypeStruct((BATCH,), jnp.float32),
      mesh=mesh,
      compiler_params=pltpu.CompilerParams(needs_layout_passes=False),
      scratch_types=[
          pltpu.VMEM((2, CHUNK), jnp.int32),
          pltpu.VMEM((2, CHUNK), jnp.int32),
          pltpu.VMEM((2, CHUNK, HIDDEN), jnp.float32),
          pltpu.VMEM((2, CHUNK, HIDDEN), jnp.float32),
          pltpu.VMEM((ROWS_PER_WORKER,), jnp.float32),
          pltpu.SemaphoreType.DMA((2,)),
          pltpu.SemaphoreType.DMA((2,)),
          pltpu.SemaphoreType.DMA((2,)),
          pltpu.SemaphoreType.DMA((2,)),
      ],
  )
  return grid_kernel(batch_user_ids, batch_item_ids, user_table, item_table)


def kernel(batch_user_ids, batch_item_ids, user_table, item_table):
  return _mf_scores(batch_user_ids, batch_item_ids, user_table, item_table)


# speedup vs baseline: 2.3434x; 1.0235x over previous
import jax
import jax.numpy as jnp
from jax import lax
from jax.experimental import pallas as pl
from jax.experimental.pallas import tpu as pltpu
from jax.experimental.pallas import tpu_sc as plsc

NUM_CORES = 2
NUM_SUBCORES = 16
NUM_WORKERS = NUM_CORES * NUM_SUBCORES
LANES = 16

BATCH = 16384
HIDDEN = 128
ROWS_PER_WORKER = BATCH // NUM_WORKERS
CHUNK = 128
NUM_CHUNKS = ROWS_PER_WORKER // CHUNK


def _mf_body(user_ids, item_ids, user_table, item_table, out_hbm,
             uidx, iidx, urows, irows, out_v,
             sem_uidx, sem_iidx, sem_urows, sem_irows):
  wid = lax.axis_index("s") * NUM_CORES + lax.axis_index("c")
  base = wid * ROWS_PER_WORKER

  def start_idx(c, p):
    off = base + c * CHUNK
    pltpu.async_copy(user_ids.at[pl.ds(off, CHUNK)], uidx.at[p], sem_uidx.at[p])
    pltpu.async_copy(item_ids.at[pl.ds(off, CHUNK)], iidx.at[p], sem_iidx.at[p])

  def wait_idx(c, p):
    off = base + c * CHUNK
    pltpu.make_async_copy(user_ids.at[pl.ds(off, CHUNK)], uidx.at[p],
                          sem_uidx.at[p]).wait()
    pltpu.make_async_copy(item_ids.at[pl.ds(off, CHUNK)], iidx.at[p],
                          sem_iidx.at[p]).wait()

  def start_rows(p):
    pltpu.async_copy(user_table.at[uidx.at[p]],
                     urows.at[pl.ds(p * CHUNK, CHUNK)], sem_urows.at[p])
    pltpu.async_copy(item_table.at[iidx.at[p]],
                     irows.at[pl.ds(p * CHUNK, CHUNK)], sem_irows.at[p])

  def wait_rows(p):
    pltpu.make_async_copy(user_table.at[uidx.at[p]],
                          urows.at[pl.ds(p * CHUNK, CHUNK)],
                          sem_urows.at[p]).wait()
    pltpu.make_async_copy(item_table.at[iidx.at[p]],
                          irows.at[pl.ds(p * CHUNK, CHUNK)],
                          sem_irows.at[p]).wait()

  def compute_chunk(cm):
    roff = lax.rem(cm, 2) * CHUNK

    @plsc.parallel_loop(0, CHUNK, unroll=2)
    def _(r):
      q = roff + r
      prods = [urows[q, pl.ds(j * LANES, LANES)] *
               irows[q, pl.ds(j * LANES, LANES)]
               for j in range(HIDDEN // LANES)]
      while len(prods) > 1:
        prods = [prods[i] + prods[i + 1] for i in range(0, len(prods), 2)]
      rowid = jnp.full((LANES,), cm * CHUNK + r, jnp.int32)
      plsc.addupdate_scatter(out_v, [rowid], prods[0])

  @plsc.parallel_loop(0, ROWS_PER_WORKER // LANES, unroll=4)
  def _(i):
    out_v[pl.ds(i * LANES, LANES)] = jnp.zeros((LANES,), jnp.float32)

  start_idx(0, 0)

  def pipe_body(c, _):
    even = lax.rem(c, 2) == 0

    @pl.when(jnp.logical_and(c < NUM_CHUNKS, even))
    def _():
      wait_idx(c, 0)
      start_rows(0)

    @pl.when(jnp.logical_and(c < NUM_CHUNKS, jnp.logical_not(even)))
    def _():
      wait_idx(c, 1)
      start_rows(1)

    @pl.when(jnp.logical_and(c > 0, even))
    def _():
      wait_rows(1)

    @pl.when(jnp.logical_and(c > 0, jnp.logical_not(even)))
    def _():
      wait_rows(0)

    @pl.when(c > 0)
    def _():
      compute_chunk(c - 1)

    @pl.when(jnp.logical_and(c + 1 < NUM_CHUNKS, even))
    def _():
      start_idx(c + 1, 1)

    @pl.when(jnp.logical_and(c + 1 < NUM_CHUNKS, jnp.logical_not(even)))
    def _():
      start_idx(c + 1, 0)

    return 0

  lax.fori_loop(0, NUM_CHUNKS + 1, pipe_body, 0)

  pltpu.sync_copy(out_v, out_hbm.at[pl.ds(base, ROWS_PER_WORKER)])


@jax.jit
def _mf_scores(batch_user_ids, batch_item_ids, user_table, item_table):
  mesh = plsc.VectorSubcoreMesh(
      core_axis_name="c", subcore_axis_name="s",
      num_cores=NUM_CORES, num_subcores=NUM_SUBCORES)
  grid_kernel = pl.kernel(
      _mf_body,
      out_type=jax.ShapeDtypeStruct((BATCH,), jnp.float32),
      mesh=mesh,
      compiler_params=pltpu.CompilerParams(needs_layout_passes=False),
      scratch_types=[
          pltpu.VMEM((2, CHUNK), jnp.int32),
          pltpu.VMEM((2, CHUNK), jnp.int32),
          pltpu.VMEM((2 * CHUNK, HIDDEN), jnp.float32),
          pltpu.VMEM((2 * CHUNK, HIDDEN), jnp.float32),
          pltpu.VMEM((ROWS_PER_WORKER,), jnp.float32),
          pltpu.SemaphoreType.DMA((2,)),
          pltpu.SemaphoreType.DMA((2,)),
          pltpu.SemaphoreType.DMA((2,)),
          pltpu.SemaphoreType.DMA((2,)),
      ],
  )
  return grid_kernel(batch_user_ids, batch_item_ids, user_table, item_table)


def kernel(batch_user_ids, batch_item_ids, user_table, item_table):
  return _mf_scores(batch_user_ids, batch_item_ids, user_table, item_table)


# speedup vs baseline: 2.3713x; 1.0119x over previous
import jax
import jax.numpy as jnp
from jax import lax
from jax.experimental import pallas as pl
from jax.experimental.pallas import tpu as pltpu
from jax.experimental.pallas import tpu_sc as plsc

NUM_CORES = 2
NUM_SUBCORES = 16
NUM_WORKERS = NUM_CORES * NUM_SUBCORES
LANES = 16

BATCH = 16384
HIDDEN = 128
ROWS_PER_WORKER = BATCH // NUM_WORKERS
CHUNK = 128
NUM_CHUNKS = ROWS_PER_WORKER // CHUNK


def _mf_body(user_ids, item_ids, user_table, item_table, out_hbm,
             uidx, iidx, urows, irows, out_v,
             sem_uidx, sem_iidx, sem_urows, sem_irows):
  wid = lax.axis_index("s") * NUM_CORES + lax.axis_index("c")
  base = wid * ROWS_PER_WORKER

  pltpu.async_copy(user_ids.at[pl.ds(base, ROWS_PER_WORKER)], uidx, sem_uidx)
  pltpu.async_copy(item_ids.at[pl.ds(base, ROWS_PER_WORKER)], iidx, sem_iidx)

  def start_rows(c, p):
    pltpu.async_copy(user_table.at[uidx.at[pl.ds(c * CHUNK, CHUNK)]],
                     urows.at[pl.ds(p * CHUNK, CHUNK)], sem_urows.at[p])
    pltpu.async_copy(item_table.at[iidx.at[pl.ds(c * CHUNK, CHUNK)]],
                     irows.at[pl.ds(p * CHUNK, CHUNK)], sem_irows.at[p])

  def wait_rows(c, p):
    pltpu.make_async_copy(user_table.at[uidx.at[pl.ds(c * CHUNK, CHUNK)]],
                          urows.at[pl.ds(p * CHUNK, CHUNK)],
                          sem_urows.at[p]).wait()
    pltpu.make_async_copy(item_table.at[iidx.at[pl.ds(c * CHUNK, CHUNK)]],
                          irows.at[pl.ds(p * CHUNK, CHUNK)],
                          sem_irows.at[p]).wait()

  def compute_chunk(cm):
    roff = lax.rem(cm, 2) * CHUNK

    @plsc.parallel_loop(0, CHUNK, unroll=2)
    def _(r):
      q = roff + r
      prods = [urows[q, pl.ds(j * LANES, LANES)] *
               irows[q, pl.ds(j * LANES, LANES)]
               for j in range(HIDDEN // LANES)]
      while len(prods) > 1:
        prods = [prods[i] + prods[i + 1] for i in range(0, len(prods), 2)]
      rowid = jnp.full((LANES,), cm * CHUNK + r, jnp.int32)
      plsc.addupdate_scatter(out_v, [rowid], prods[0])

  @plsc.parallel_loop(0, ROWS_PER_WORKER // LANES, unroll=4)
  def _(i):
    out_v[pl.ds(i * LANES, LANES)] = jnp.zeros((LANES,), jnp.float32)

  pltpu.make_async_copy(user_ids.at[pl.ds(base, ROWS_PER_WORKER)], uidx,
                        sem_uidx).wait()
  pltpu.make_async_copy(item_ids.at[pl.ds(base, ROWS_PER_WORKER)], iidx,
                        sem_iidx).wait()

  def pipe_body(c, _):
    even = lax.rem(c, 2) == 0

    @pl.when(jnp.logical_and(c < NUM_CHUNKS, even))
    def _():
      start_rows(c, 0)

    @pl.when(jnp.logical_and(c < NUM_CHUNKS, jnp.logical_not(even)))
    def _():
      start_rows(c, 1)

    @pl.when(jnp.logical_and(c > 0, even))
    def _():
      wait_rows(c - 1, 1)

    @pl.when(jnp.logical_and(c > 0, jnp.logical_not(even)))
    def _():
      wait_rows(c - 1, 0)

    @pl.when(c > 0)
    def _():
      compute_chunk(c - 1)

    return 0

  lax.fori_loop(0, NUM_CHUNKS + 1, pipe_body, 0)

  pltpu.sync_copy(out_v, out_hbm.at[pl.ds(base, ROWS_PER_WORKER)])


@jax.jit
def _mf_scores(batch_user_ids, batch_item_ids, user_table, item_table):
  mesh = plsc.VectorSubcoreMesh(
      core_axis_name="c", subcore_axis_name="s",
      num_cores=NUM_CORES, num_subcores=NUM_SUBCORES)
  grid_kernel = pl.kernel(
      _mf_body,
      out_type=jax.ShapeDtypeStruct((BATCH,), jnp.float32),
      mesh=mesh,
      compiler_params=pltpu.CompilerParams(needs_layout_passes=False),
      scratch_types=[
          pltpu.VMEM((ROWS_PER_WORKER,), jnp.int32),
          pltpu.VMEM((ROWS_PER_WORKER,), jnp.int32),
          pltpu.VMEM((2 * CHUNK, HIDDEN), jnp.float32),
          pltpu.VMEM((2 * CHUNK, HIDDEN), jnp.float32),
          pltpu.VMEM((ROWS_PER_WORKER,), jnp.float32),
          pltpu.SemaphoreType.DMA,
          pltpu.SemaphoreType.DMA,
          pltpu.SemaphoreType.DMA((2,)),
          pltpu.SemaphoreType.DMA((2,)),
      ],
  )
  return grid_kernel(batch_user_ids, batch_item_ids, user_table, item_table)


def kernel(batch_user_ids, batch_item_ids, user_table, item_table):
  return _mf_scores(batch_user_ids, batch_item_ids, user_table, item_table)


# speedup vs baseline: 2.8714x; 1.2109x over previous
import jax
import jax.numpy as jnp
from jax import lax
from jax.experimental import pallas as pl
from jax.experimental.pallas import tpu as pltpu
from jax.experimental.pallas import tpu_sc as plsc

NUM_CORES = 2
NUM_SUBCORES = 16
NUM_WORKERS = NUM_CORES * NUM_SUBCORES
LANES = 16

BATCH = 16384
HIDDEN = 128
ROWS_PER_WORKER = BATCH // NUM_WORKERS
CHUNK = 128
NUM_CHUNKS = ROWS_PER_WORKER // CHUNK


def _mf_body(user_ids, item_ids, user_table, item_table, out_hbm,
             uidx, iidx, urows, irows, out_v,
             sem_uidx, sem_iidx, sem_urows, sem_irows):
  wid = lax.axis_index("s") * NUM_CORES + lax.axis_index("c")
  base = wid * ROWS_PER_WORKER

  pltpu.async_copy(user_ids.at[pl.ds(base, ROWS_PER_WORKER)], uidx, sem_uidx)
  pltpu.async_copy(item_ids.at[pl.ds(base, ROWS_PER_WORKER)], iidx, sem_iidx)

  def start_rows(c, p):
    pltpu.async_copy(user_table.at[uidx.at[pl.ds(c * CHUNK, CHUNK)]],
                     urows.at[pl.ds(p * CHUNK, CHUNK)], sem_urows.at[p])
    pltpu.async_copy(item_table.at[iidx.at[pl.ds(c * CHUNK, CHUNK)]],
                     irows.at[pl.ds(p * CHUNK, CHUNK)], sem_irows.at[p])

  def wait_rows(c, p):
    pltpu.make_async_copy(user_table.at[uidx.at[pl.ds(c * CHUNK, CHUNK)]],
                          urows.at[pl.ds(p * CHUNK, CHUNK)],
                          sem_urows.at[p]).wait()
    pltpu.make_async_copy(item_table.at[iidx.at[pl.ds(c * CHUNK, CHUNK)]],
                          irows.at[pl.ds(p * CHUNK, CHUNK)],
                          sem_irows.at[p]).wait()

  lane = lax.iota(jnp.int32, LANES)
  perms = [jnp.bitwise_xor(lane, k) for k in (8, 4, 2, 1)]
  mask0 = lane == 0

  def compute_chunk(cm):
    roff = lax.rem(cm, 2) * CHUNK

    @plsc.parallel_loop(0, CHUNK, unroll=2)
    def _(r):
      q = roff + r
      prods = [urows[q, pl.ds(j * LANES, LANES)] *
               irows[q, pl.ds(j * LANES, LANES)]
               for j in range(HIDDEN // LANES)]
      while len(prods) > 1:
        prods = [prods[i] + prods[i + 1] for i in range(0, len(prods), 2)]
      s = prods[0]
      for pm in perms:
        s = s + jnp.take_along_axis(s, pm, axis=0)
      rowid = jnp.full((LANES,), cm * CHUNK + r, jnp.int32)
      plsc.store_scatter(out_v, [rowid], s, mask=mask0)

  pltpu.make_async_copy(user_ids.at[pl.ds(base, ROWS_PER_WORKER)], uidx,
                        sem_uidx).wait()
  pltpu.make_async_copy(item_ids.at[pl.ds(base, ROWS_PER_WORKER)], iidx,
                        sem_iidx).wait()

  def pipe_body(c, _):
    even = lax.rem(c, 2) == 0

    @pl.when(jnp.logical_and(c < NUM_CHUNKS, even))
    def _():
      start_rows(c, 0)

    @pl.when(jnp.logical_and(c < NUM_CHUNKS, jnp.logical_not(even)))
    def _():
      start_rows(c, 1)

    @pl.when(jnp.logical_and(c > 0, even))
    def _():
      wait_rows(c - 1, 1)

    @pl.when(jnp.logical_and(c > 0, jnp.logical_not(even)))
    def _():
      wait_rows(c - 1, 0)

    @pl.when(c > 0)
    def _():
      compute_chunk(c - 1)

    return 0

  lax.fori_loop(0, NUM_CHUNKS + 1, pipe_body, 0)

  pltpu.sync_copy(out_v, out_hbm.at[pl.ds(base, ROWS_PER_WORKER)])


@jax.jit
def _mf_scores(batch_user_ids, batch_item_ids, user_table, item_table):
  mesh = plsc.VectorSubcoreMesh(
      core_axis_name="c", subcore_axis_name="s",
      num_cores=NUM_CORES, num_subcores=NUM_SUBCORES)
  grid_kernel = pl.kernel(
      _mf_body,
      out_type=jax.ShapeDtypeStruct((BATCH,), jnp.float32),
      mesh=mesh,
      compiler_params=pltpu.CompilerParams(needs_layout_passes=False),
      scratch_types=[
          pltpu.VMEM((ROWS_PER_WORKER,), jnp.int32),
          pltpu.VMEM((ROWS_PER_WORKER,), jnp.int32),
          pltpu.VMEM((2 * CHUNK, HIDDEN), jnp.float32),
          pltpu.VMEM((2 * CHUNK, HIDDEN), jnp.float32),
          pltpu.VMEM((ROWS_PER_WORKER,), jnp.float32),
          pltpu.SemaphoreType.DMA,
          pltpu.SemaphoreType.DMA,
          pltpu.SemaphoreType.DMA((2,)),
          pltpu.SemaphoreType.DMA((2,)),
      ],
  )
  return grid_kernel(batch_user_ids, batch_item_ids, user_table, item_table)


def kernel(batch_user_ids, batch_item_ids, user_table, item_table):
  return _mf_scores(batch_user_ids, batch_item_ids, user_table, item_table)


# speedup vs baseline: 2.9243x; 1.0184x over previous
import jax
import jax.numpy as jnp
from jax import lax
from jax.experimental import pallas as pl
from jax.experimental.pallas import tpu as pltpu
from jax.experimental.pallas import tpu_sc as plsc

NUM_CORES = 2
NUM_SUBCORES = 16
NUM_WORKERS = NUM_CORES * NUM_SUBCORES
LANES = 16

BATCH = 16384
HIDDEN = 128
ROWS_PER_WORKER = BATCH // NUM_WORKERS
CHUNK = 128
NUM_CHUNKS = ROWS_PER_WORKER // CHUNK
NBUF = 3
LAG = 2


def _mf_body(user_ids, item_ids, user_table, item_table, out_hbm,
             uidx, iidx, urows, irows, out_v,
             sem_uidx, sem_iidx, sem_urows, sem_irows):
  wid = lax.axis_index("s") * NUM_CORES + lax.axis_index("c")
  base = wid * ROWS_PER_WORKER

  pltpu.async_copy(user_ids.at[pl.ds(base, ROWS_PER_WORKER)], uidx, sem_uidx)
  pltpu.async_copy(item_ids.at[pl.ds(base, ROWS_PER_WORKER)], iidx, sem_iidx)

  def start_rows(c, p):
    pltpu.async_copy(user_table.at[uidx.at[pl.ds(c * CHUNK, CHUNK)]],
                     urows.at[pl.ds(p * CHUNK, CHUNK)], sem_urows.at[p])
    pltpu.async_copy(item_table.at[iidx.at[pl.ds(c * CHUNK, CHUNK)]],
                     irows.at[pl.ds(p * CHUNK, CHUNK)], sem_irows.at[p])

  def wait_rows(c, p):
    pltpu.make_async_copy(user_table.at[uidx.at[pl.ds(c * CHUNK, CHUNK)]],
                          urows.at[pl.ds(p * CHUNK, CHUNK)],
                          sem_urows.at[p]).wait()
    pltpu.make_async_copy(item_table.at[iidx.at[pl.ds(c * CHUNK, CHUNK)]],
                          irows.at[pl.ds(p * CHUNK, CHUNK)],
                          sem_irows.at[p]).wait()

  lane = lax.iota(jnp.int32, LANES)
  perms = [jnp.bitwise_xor(lane, k) for k in (8, 4, 2, 1)]
  mask0 = lane == 0

  def compute_chunk(cm):
    roff = lax.rem(cm, NBUF) * CHUNK

    @plsc.parallel_loop(0, CHUNK, unroll=2)
    def _(r):
      q = roff + r
      prods = [urows[q, pl.ds(j * LANES, LANES)] *
               irows[q, pl.ds(j * LANES, LANES)]
               for j in range(HIDDEN // LANES)]
      while len(prods) > 1:
        prods = [prods[i] + prods[i + 1] for i in range(0, len(prods), 2)]
      s = prods[0]
      for pm in perms:
        s = s + jnp.take_along_axis(s, pm, axis=0)
      rowid = jnp.full((LANES,), cm * CHUNK + r, jnp.int32)
      plsc.store_scatter(out_v, [rowid], s, mask=mask0)

  pltpu.make_async_copy(user_ids.at[pl.ds(base, ROWS_PER_WORKER)], uidx,
                        sem_uidx).wait()
  pltpu.make_async_copy(item_ids.at[pl.ds(base, ROWS_PER_WORKER)], iidx,
                        sem_iidx).wait()

  def pipe_body(c, _):
    ms = lax.rem(c, NBUF)
    mw = lax.rem(c - LAG + NBUF, NBUF)

    for b in range(NBUF):
      @pl.when(jnp.logical_and(c < NUM_CHUNKS, ms == b))
      def _(b=b):
        start_rows(c, b)

    for b in range(NBUF):
      @pl.when(jnp.logical_and(c >= LAG, mw == b))
      def _(b=b):
        wait_rows(c - LAG, b)

    @pl.when(c >= LAG)
    def _():
      compute_chunk(c - LAG)

    return 0

  lax.fori_loop(0, NUM_CHUNKS + LAG, pipe_body, 0)

  pltpu.sync_copy(out_v, out_hbm.at[pl.ds(base, ROWS_PER_WORKER)])


@jax.jit
def _mf_scores(batch_user_ids, batch_item_ids, user_table, item_table):
  mesh = plsc.VectorSubcoreMesh(
      core_axis_name="c", subcore_axis_name="s",
      num_cores=NUM_CORES, num_subcores=NUM_SUBCORES)
  grid_kernel = pl.kernel(
      _mf_body,
      out_type=jax.ShapeDtypeStruct((BATCH,), jnp.float32),
      mesh=mesh,
      compiler_params=pltpu.CompilerParams(needs_layout_passes=False),
      scratch_types=[
          pltpu.VMEM((ROWS_PER_WORKER,), jnp.int32),
          pltpu.VMEM((ROWS_PER_WORKER,), jnp.int32),
          pltpu.VMEM((NBUF * CHUNK, HIDDEN), jnp.float32),
          pltpu.VMEM((NBUF * CHUNK, HIDDEN), jnp.float32),
          pltpu.VMEM((ROWS_PER_WORKER,), jnp.float32),
          pltpu.SemaphoreType.DMA,
          pltpu.SemaphoreType.DMA,
          pltpu.SemaphoreType.DMA((NBUF,)),
          pltpu.SemaphoreType.DMA((NBUF,)),
      ],
  )
  return grid_kernel(batch_user_ids, batch_item_ids, user_table, item_table)


def kernel(batch_user_ids, batch_item_ids, user_table, item_table):
  return _mf_scores(batch_user_ids, batch_item_ids, user_table, item_table)
